# Initial kernel scaffold; baseline (speedup 1.0000x reference)
import jax
import jax.numpy as jnp
from jax.experimental import pallas as pl


def kernel(t, x, grid_points, t_edges, grid_adjoints, grid_counts):
    raise NotImplementedError("write your pallas kernel here")


# speedup vs baseline: 1.3342x; 1.3342x over previous
import jax
import jax.numpy as jnp
from jax.experimental import pallas as pl

_BANDWIDTH = 0.5
_BB = 512
_GPAD = 2560
_TPAD = 32


def _body(t_ref, x_ref, g_ref, e_ref, cnt_ref, a0_ref, a1_ref, o_ref):
    x0 = x_ref[:, 0:1]
    x1 = x_ref[:, 1:2]
    gx = g_ref[0:1, :]
    gy = g_ref[1:2, :]
    x0b = x0.astype(jnp.bfloat16).astype(jnp.float32)
    x1b = x1.astype(jnp.bfloat16).astype(jnp.float32)
    gxb = gx.astype(jnp.bfloat16).astype(jnp.float32)
    gyb = gy.astype(jnp.bfloat16).astype(jnp.float32)
    xdotg = x0b * gxb + x1b * gyb
    x2 = x0 * x0 + x1 * x1
    g2 = gx * gx + gy * gy
    sq = jnp.maximum(x2 + g2 - 2.0 * xdotg, 0.0)
    w = jnp.exp(sq * (-1.0 / (2.0 * _BANDWIDTH ** 2)))

    m = (cnt_ref[...] > 0.0).astype(jnp.float32)
    hi = jax.lax.Precision.HIGHEST
    yd = jax.lax.dot(w, m, precision=hi, preferred_element_type=jnp.float32)
    yn0 = jax.lax.dot(w, a0_ref[...] * m, precision=hi,
                      preferred_element_type=jnp.float32)
    yn1 = jax.lax.dot(w, a1_ref[...] * m, precision=hi,
                      preferred_element_type=jnp.float32)

    tt = t_ref[...]
    e = e_ref[0:1, :]
    bins = jnp.sum((e < tt).astype(jnp.int32), axis=1, keepdims=True)
    lane = jax.lax.broadcasted_iota(jnp.int32, (tt.shape[0], _TPAD), 1)
    oh = (lane == bins).astype(jnp.float32)

    den = jnp.sum(yd * oh, axis=1, keepdims=True) + 1e-10
    n0 = jnp.sum(yn0 * oh, axis=1, keepdims=True)
    n1 = jnp.sum(yn1 * oh, axis=1, keepdims=True)
    o_ref[:, 0:1] = -(n0 / den)
    o_ref[:, 1:2] = -(n1 / den)


def kernel(t, x, grid_points, t_edges, grid_adjoints, grid_counts):
    B = x.shape[0]
    G = grid_points.shape[0]
    T = grid_counts.shape[0]

    g = jnp.zeros((8, _GPAD), jnp.float32)
    g = g.at[0, :G].set(grid_points[:, 0]).at[1, :G].set(grid_points[:, 1])
    e = jnp.full((8, _TPAD), jnp.inf, jnp.float32)
    e = e.at[0, : T - 1].set(t_edges[1:T])
    cnt = jnp.zeros((_GPAD, _TPAD), jnp.float32).at[:G, :T].set(grid_counts.T)
    a0 = jnp.zeros((_GPAD, _TPAD), jnp.float32)
    a0 = a0.at[:G, :T].set(grid_adjoints[:, :, 0].T)
    a1 = jnp.zeros((_GPAD, _TPAD), jnp.float32)
    a1 = a1.at[:G, :T].set(grid_adjoints[:, :, 1].T)

    return pl.pallas_call(
        _body,
        grid=(B // _BB,),
        in_specs=[
            pl.BlockSpec((_BB, 1), lambda i: (i, 0)),
            pl.BlockSpec((_BB, 2), lambda i: (i, 0)),
            pl.BlockSpec((8, _GPAD), lambda i: (0, 0)),
            pl.BlockSpec((8, _TPAD), lambda i: (0, 0)),
            pl.BlockSpec((_GPAD, _TPAD), lambda i: (0, 0)),
            pl.BlockSpec((_GPAD, _TPAD), lambda i: (0, 0)),
            pl.BlockSpec((_GPAD, _TPAD), lambda i: (0, 0)),
        ],
        out_specs=pl.BlockSpec((_BB, 2), lambda i: (i, 0)),
        out_shape=jax.ShapeDtypeStruct((B, 2), jnp.float32),
    )(t, x, g, e, cnt, a0, a1)


# speedup vs baseline: 4.4089x; 3.3044x over previous
import jax
import jax.numpy as jnp
from jax.experimental import pallas as pl
from jax.experimental.pallas import tpu as pltpu

_BANDWIDTH = 0.5
_BB = 512
_GPAD = 2560
_TPAD = 32


def _body(t_ref, x_ref, g_ref, e_ref, cnt_ref, a0_ref, a1_ref, o_ref,
          pk_ref):
    @pl.when(pl.program_id(0) == 0)
    def _():
        m = (cnt_ref[...] > 0.0).astype(jnp.float32)
        pk_ref[...] = jnp.concatenate(
            [m, a0_ref[...] * m, a1_ref[...] * m], axis=1)

    xdotg = jax.lax.dot(x_ref[...], g_ref[...],
                        preferred_element_type=jnp.float32)
    x0 = x_ref[:, 0:1]
    x1 = x_ref[:, 1:2]
    gx = g_ref[0:1, :]
    gy = g_ref[1:2, :]
    x2 = x0 * x0 + x1 * x1
    g2 = gx * gx + gy * gy
    sq = jnp.maximum(x2 + g2 - 2.0 * xdotg, 0.0)
    w = jnp.exp(sq * (-1.0 / (2.0 * _BANDWIDTH ** 2)))

    y = jax.lax.dot(w, pk_ref[...],
                    preferred_element_type=jnp.float32)

    tt = t_ref[...]
    e = e_ref[0:1, :]
    bins = jnp.sum((e < tt).astype(jnp.int32), axis=1, keepdims=True)
    lane = jax.lax.broadcasted_iota(jnp.int32, (tt.shape[0], _TPAD), 1)
    oh = (lane == bins).astype(jnp.float32)

    den = jnp.sum(y[:, 0:_TPAD] * oh, axis=1, keepdims=True) + 1e-10
    n0 = jnp.sum(y[:, _TPAD:2 * _TPAD] * oh, axis=1, keepdims=True)
    n1 = jnp.sum(y[:, 2 * _TPAD:3 * _TPAD] * oh, axis=1, keepdims=True)
    o_ref[:, 0:1] = -(n0 / den)
    o_ref[:, 1:2] = -(n1 / den)


def kernel(t, x, grid_points, t_edges, grid_adjoints, grid_counts):
    B = x.shape[0]
    G = grid_points.shape[0]
    T = grid_counts.shape[0]

    xp = jnp.zeros((B, 8), jnp.float32).at[:, :2].set(x)
    g = jnp.zeros((8, _GPAD), jnp.float32)
    g = g.at[0, :G].set(grid_points[:, 0]).at[1, :G].set(grid_points[:, 1])
    e = jnp.full((8, _TPAD), jnp.inf, jnp.float32)
    e = e.at[0, : T - 1].set(t_edges[1:T])
    cnt = jnp.zeros((_GPAD, _TPAD), jnp.float32).at[:G, :T].set(grid_counts.T)
    a0 = jnp.zeros((_GPAD, _TPAD), jnp.float32)
    a0 = a0.at[:G, :T].set(grid_adjoints[:, :, 0].T)
    a1 = jnp.zeros((_GPAD, _TPAD), jnp.float32)
    a1 = a1.at[:G, :T].set(grid_adjoints[:, :, 1].T)

    return pl.pallas_call(
        _body,
        grid=(B // _BB,),
        in_specs=[
            pl.BlockSpec((_BB, 1), lambda i: (i, 0)),
            pl.BlockSpec((_BB, 8), lambda i: (i, 0)),
            pl.BlockSpec((8, _GPAD), lambda i: (0, 0)),
            pl.BlockSpec((8, _TPAD), lambda i: (0, 0)),
            pl.BlockSpec((_GPAD, _TPAD), lambda i: (0, 0)),
            pl.BlockSpec((_GPAD, _TPAD), lambda i: (0, 0)),
            pl.BlockSpec((_GPAD, _TPAD), lambda i: (0, 0)),
        ],
        out_specs=pl.BlockSpec((_BB, 2), lambda i: (i, 0)),
        out_shape=jax.ShapeDtypeStruct((B, 2), jnp.float32),
        scratch_shapes=[pltpu.VMEM((_GPAD, 3 * _TPAD), jnp.float32)],
    )(t, xp, g, e, cnt, a0, a1)
